# Initial kernel scaffold; baseline (speedup 1.0000x reference)
import jax
import jax.numpy as jnp
from jax.experimental import pallas as pl


def kernel(x, edge_index, edge_type, batch, comp1, bases1, root1, bias1, comp2, bases2, root2, bias2, bn_gamma, bn_beta, fc1_w, fc1_b, fc2_w, fc2_b):
    raise NotImplementedError("write your pallas kernel here")


# speedup vs baseline: 5.9661x; 5.9661x over previous
import functools
import math

import jax
import jax.numpy as jnp
from jax import lax
from jax.experimental import pallas as pl
from jax.experimental.pallas import tpu as pltpu
from jax.experimental.pallas import tpu_sc as plsc

N = 50000
E = 800000
R = 4
NBASE = 30
FIN = 86
H0 = 56
G = 32
HP = 64
HIST = 204800
NC, NS = 2, 16
NW = NC * NS
HALF = N // 2
ACC_ROWS = HALF + 24
ESH = E // NW
ESC = E // NS


def _i32(v):
    return jnp.full((16,), v, jnp.int32)


def _f32(v):
    return jnp.full((16,), v, jnp.float32)


def _mesh():
    return plsc.VectorSubcoreMesh(core_axis_name="c", subcore_axis_name="s")


_SC_PARAMS = pltpu.CompilerParams(use_tc_tiling_on_sc=False)


def _sc_hist(dstv, etype):
    @functools.partial(
        pl.kernel, mesh=_mesh(), compiler_params=_SC_PARAMS,
        out_type=jax.ShapeDtypeStruct((NC, HIST), jnp.float32),
        scratch_types=[
            pltpu.VMEM((2048,), jnp.float32),
            pltpu.VMEM((256,), jnp.int32),
            pltpu.VMEM((256,), jnp.int32),
            pltpu.VMEM((256,), jnp.int32),
            pltpu.VMEM((256,), jnp.float32),
            pltpu.SemaphoreType.DMA,
            pltpu.VMEM_SHARED((HIST,), jnp.float32),
        ],
    )
    def k(dst_hbm, et_hbm, hist_hbm, zbuf, dbuf, tbuf, ibuf, ones, sem,
          hist_sh):
        c = lax.axis_index("c")
        s = lax.axis_index("s")
        wid = s * NC + c

        def zf(i, _):
            zbuf[pl.ds(i * 16, 16)] = _f32(0.0)
            return 0

        lax.fori_loop(0, 128, zf, 0, unroll=False)

        def of(i, _):
            ones[pl.ds(i * 16, 16)] = _f32(1.0)
            return 0

        lax.fori_loop(0, 16, of, 0, unroll=False)

        base = s * 12800
        for j in range(6):
            pltpu.sync_copy(zbuf, hist_sh.at[pl.ds(base + j * 2048, 2048)])
        pltpu.sync_copy(zbuf.at[pl.ds(0, 512)],
                        hist_sh.at[pl.ds(base + 12288, 512)])
        plsc.subcore_barrier()

        eb = wid * ESH

        def batch(b, _):
            off = eb + b * 256
            pltpu.sync_copy(dst_hbm.at[pl.ds(off, 256)], dbuf)
            pltpu.sync_copy(et_hbm.at[pl.ds(off, 256)], tbuf)

            def ch(g, _):
                sl = pl.ds(g * 16, 16)
                ibuf[sl] = tbuf[sl] * _i32(N) + dbuf[sl]
                return 0

            lax.fori_loop(0, 16, ch, 0, unroll=False)
            pltpu.sync_copy(ones, hist_sh.at[ibuf], add=True)
            return 0

        lax.fori_loop(0, 97, batch, 0, unroll=False)

        off = eb + 97 * 256
        pltpu.sync_copy(dst_hbm.at[pl.ds(off, 168)], dbuf.at[pl.ds(0, 168)])
        pltpu.sync_copy(et_hbm.at[pl.ds(off, 168)], tbuf.at[pl.ds(0, 168)])
        lanes = lax.iota(jnp.int32, 16)
        trash = _i32(200000) + lanes + jnp.full((16,), wid * 16, jnp.int32)
        for g in range(11):
            sl = pl.ds(g * 16, 16)
            v = tbuf[sl] * _i32(N) + dbuf[sl]
            if g == 10:
                v = jnp.where(lanes < _i32(8), v, trash)
            ibuf[sl] = v
        for g in range(11, 16):
            ibuf[pl.ds(g * 16, 16)] = trash
        pltpu.sync_copy(ones, hist_sh.at[ibuf], add=True)
        plsc.subcore_barrier()

        for j in range(6):
            pltpu.sync_copy(hist_sh.at[pl.ds(base + j * 2048, 2048)], zbuf)
            pltpu.sync_copy(zbuf, hist_hbm.at[c, pl.ds(base + j * 2048, 2048)])
        pltpu.sync_copy(hist_sh.at[pl.ds(base + 12288, 512)],
                        zbuf.at[pl.ds(0, 512)])
        pltpu.sync_copy(zbuf.at[pl.ds(0, 512)],
                        hist_hbm.at[c, pl.ds(base + 12288, 512)])

    return k(dstv, etype)


def _sc_edge_prep(hist, src, dstv, etype):
    @functools.partial(
        pl.kernel, mesh=_mesh(), compiler_params=_SC_PARAMS,
        out_type=(jax.ShapeDtypeStruct((E,), jnp.int32),
                  jax.ShapeDtypeStruct((E,), jnp.float32)),
        scratch_types=[
            pltpu.VMEM((2048,), jnp.float32),
            pltpu.VMEM((2048,), jnp.float32),
            pltpu.VMEM((256,), jnp.int32),
            pltpu.VMEM((256,), jnp.int32),
            pltpu.VMEM((256,), jnp.int32),
            pltpu.VMEM((256,), jnp.int32),
            pltpu.VMEM((256,), jnp.int32),
            pltpu.VMEM((256,), jnp.float32),
            pltpu.SemaphoreType.DMA,
            pltpu.VMEM_SHARED((HIST,), jnp.float32),
        ],
    )
    def k(hist_hbm, src_hbm, dst_hbm, et_hbm, gidx_hbm, scale_hbm,
          h0buf, h1buf, sbuf, dbuf, tbuf, gbuf, ibuf, scbuf, sem, recip_sh):
        c = lax.axis_index("c")
        s = lax.axis_index("s")
        wid = s * NC + c
        base = s * 12800
        for j in range(7):
            ln = 2048 if j < 6 else 512
            o = base + j * 2048
            pltpu.sync_copy(hist_hbm.at[0, pl.ds(o, ln)],
                            h0buf.at[pl.ds(0, ln)])
            pltpu.sync_copy(hist_hbm.at[1, pl.ds(o, ln)],
                            h1buf.at[pl.ds(0, ln)])

            def rc(i, _):
                sl = pl.ds(i * 16, 16)
                h = jnp.maximum(h0buf[sl] + h1buf[sl], _f32(1.0))
                h0buf[sl] = _f32(1.0) / h
                return 0

            lax.fori_loop(0, ln // 16, rc, 0, unroll=False)
            pltpu.sync_copy(h0buf.at[pl.ds(0, ln)], recip_sh.at[pl.ds(o, ln)])
        plsc.subcore_barrier()

        eb = wid * ESH
        lanes = lax.iota(jnp.int32, 16)

        def do_batch(off, nv):
            nch = (nv + 15) // 16
            pltpu.sync_copy(src_hbm.at[pl.ds(off, nv)], sbuf.at[pl.ds(0, nv)])
            pltpu.sync_copy(dst_hbm.at[pl.ds(off, nv)], dbuf.at[pl.ds(0, nv)])
            pltpu.sync_copy(et_hbm.at[pl.ds(off, nv)], tbuf.at[pl.ds(0, nv)])

            def ch(g, _):
                sl = pl.ds(g * 16, 16)
                t = tbuf[sl]
                gbuf[sl] = t * _i32(N) + sbuf[sl]
                ibuf[sl] = t * _i32(N) + dbuf[sl]
                return 0

            lax.fori_loop(0, nch - 1, ch, 0, unroll=False)
            g = nch - 1
            rem = nv - g * 16
            sl = pl.ds(g * 16, 16)
            t = tbuf[sl]
            gv = t * _i32(N) + sbuf[sl]
            iv = t * _i32(N) + dbuf[sl]
            if rem < 16:
                m = lanes < _i32(rem)
                gv = jnp.where(m, gv, _i32(0))
                iv = jnp.where(m, iv, _i32(0))
            gbuf[sl] = gv
            ibuf[sl] = iv
            glen = nch * 16
            pltpu.sync_copy(recip_sh.at[ibuf.at[pl.ds(0, glen)]],
                            scbuf.at[pl.ds(0, glen)])
            pltpu.sync_copy(gbuf.at[pl.ds(0, nv)],
                            gidx_hbm.at[pl.ds(off, nv)])
            pltpu.sync_copy(scbuf.at[pl.ds(0, nv)],
                            scale_hbm.at[pl.ds(off, nv)])

        def batch(b, _):
            do_batch(eb + b * 256, 256)
            return 0

        lax.fori_loop(0, 97, batch, 0, unroll=False)
        do_batch(eb + 97 * 256, 168)

    return k(hist, src, dstv, etype)


def _sc_agg(hbflat, z, gidx, scale, dstv):
    @functools.partial(
        pl.kernel, mesh=_mesh(), compiler_params=_SC_PARAMS,
        out_type=jax.ShapeDtypeStruct((N, HP), jnp.float32),
        scratch_types=[
            pltpu.VMEM((256,), jnp.int32),
            pltpu.VMEM((256,), jnp.int32),
            pltpu.VMEM((256,), jnp.int32),
            pltpu.VMEM((256,), jnp.float32),
            pltpu.VMEM((256, HP), jnp.float32),
            pltpu.VMEM((128, HP), jnp.float32),
            pltpu.SemaphoreType.DMA,
            pltpu.VMEM_SHARED((ACC_ROWS, HP), jnp.float32),
        ],
    )
    def k(hb_hbm, z_hbm, gidx_hbm, scale_hbm, dst_hbm, o_hbm,
          gbuf, dbuf, lbuf, scbuf, rows, stage, sem, acc_sh):
        c = lax.axis_index("c")
        s = lax.axis_index("s")
        lo = c * HALF
        r0 = s * 1568

        def initc(j, ln):
            pltpu.sync_copy(z_hbm.at[pl.ds(lo + r0 + j * 128, ln)],
                            stage.at[pl.ds(0, ln)])
            pltpu.sync_copy(stage.at[pl.ds(0, ln)],
                            acc_sh.at[pl.ds(r0 + j * 128, ln)])

        @pl.when(s < 15)
        def _():
            for j in range(12):
                initc(j, 128)
            initc(12, 32)

        @pl.when(s == 15)
        def _():
            for j in range(11):
                initc(j, 128)
            initc(11, 72)

        plsc.subcore_barrier()
        eb = s * ESC

        def do_batch(off, nv):
            nch = nv // 16
            pltpu.sync_copy(gidx_hbm.at[pl.ds(off, nv)],
                            gbuf.at[pl.ds(0, nv)])
            pltpu.sync_copy(dst_hbm.at[pl.ds(off, nv)],
                            dbuf.at[pl.ds(0, nv)])
            pltpu.sync_copy(scale_hbm.at[pl.ds(off, nv)],
                            scbuf.at[pl.ds(0, nv)])

            def prep(g, _):
                sl = pl.ds(g * 16, 16)
                d = dbuf[sl]
                inh = (d >= _i32(lo)) & (d < _i32(lo + HALF))
                trash = jnp.full((16,), HALF + g, jnp.int32)
                lbuf[sl] = jnp.where(inh, d - _i32(lo), trash)
                scbuf[sl] = jnp.where(inh, scbuf[sl], _f32(0.0))
                return 0

            lax.fori_loop(0, nch, prep, 0, unroll=False)
            pltpu.async_copy(hb_hbm.at[gbuf.at[pl.ds(0, nv)]],
                             rows.at[pl.ds(0, nv), :], sem).wait()

            def mul(g, _):
                sv16 = scbuf[pl.ds(g * 16, 16)]
                for j in range(16):
                    e = g * 16 + j
                    sv = jnp.full((16,), sv16[j], jnp.float32)
                    for kk in range(4):
                        sl2 = pl.ds(kk * 16, 16)
                        rows[e, sl2] = rows[e, sl2] * sv
                return 0

            lax.fori_loop(0, nch, mul, 0, unroll=False)
            pltpu.sync_copy(rows.at[pl.ds(0, nv), :],
                            acc_sh.at[lbuf.at[pl.ds(0, nv)]], add=True)

        def batch(b, _):
            do_batch(eb + b * 256, 256)
            return 0

        lax.fori_loop(0, 195, batch, 0, unroll=False)
        do_batch(eb + 195 * 256, 80)
        plsc.subcore_barrier()

        def drain(j, ln):
            pltpu.sync_copy(acc_sh.at[pl.ds(r0 + j * 128, ln)],
                            stage.at[pl.ds(0, ln)])
            pltpu.sync_copy(stage.at[pl.ds(0, ln)],
                            o_hbm.at[pl.ds(lo + r0 + j * 128, ln)])

        @pl.when(s < 15)
        def _():
            for j in range(12):
                drain(j, 128)
            drain(12, 32)

        @pl.when(s == 15)
        def _():
            for j in range(11):
                drain(j, 128)
            drain(11, 72)

    return k(hbflat, z, gidx, scale, dstv)


def _tc_small_mm(a, b):
    def body(a_ref, b_ref, o_ref):
        o_ref[...] = lax.dot_general(
            a_ref[...], b_ref[...], (((1,), (0,)), ((), ())),
            precision=lax.Precision.HIGHEST,
            preferred_element_type=jnp.float32)

    return pl.pallas_call(
        body,
        out_shape=jax.ShapeDtypeStruct((a.shape[0], b.shape[1]), jnp.float32),
    )(a, b)


def _tc_matmul(x, wcat, bcat, relu_in):
    n, fin = x.shape
    bn = 2000
    nb = n // bn

    def body(x_ref, w_ref, b_ref, z_ref, hb_ref):
        xb = x_ref[...]
        if relu_in:
            xb = jnp.maximum(xb, 0.0)
        y = lax.dot_general(xb, w_ref[...], (((1,), (0,)), ((), ())),
                            precision=lax.Precision.HIGHEST,
                            preferred_element_type=jnp.float32)
        y = y + b_ref[0, :][None, :]
        z_ref[...] = y[:, :HP]
        for r in range(R):
            hb_ref[r] = y[:, HP * (r + 1):HP * (r + 2)]

    return pl.pallas_call(
        body,
        grid=(nb,),
        in_specs=[pl.BlockSpec((bn, fin), lambda i: (i, 0)),
                  pl.BlockSpec((fin, 5 * HP), lambda i: (0, 0)),
                  pl.BlockSpec((1, 5 * HP), lambda i: (0, 0))],
        out_specs=[pl.BlockSpec((bn, HP), lambda i: (i, 0)),
                   pl.BlockSpec((R, bn, HP), lambda i: (0, i, 0))],
        out_shape=[jax.ShapeDtypeStruct((n, HP), jnp.float32),
                   jax.ShapeDtypeStruct((R, n, HP), jnp.float32)],
    )(x, wcat, bcat)


def _tc_pool_head(o2, batch3, gam, bet, w1, b1, w2, b2):
    bn = 1000
    nb = N // bn
    inv_std = 1.0 / math.sqrt(1.0 + 1e-5)

    def body(x_ref, bt_ref, g_ref, be_ref, w1_ref, b1_ref, w2_ref, b2_ref,
             pool_ref, out_ref):
        i = pl.program_id(0)
        bb = bt_ref[0, 0, :]
        oh = (lax.broadcasted_iota(jnp.int32, (G, bn), 0)
              == bb[None, :]).astype(jnp.float32)
        part = lax.dot_general(oh, x_ref[...], (((1,), (0,)), ((), ())),
                               precision=lax.Precision.HIGHEST,
                               preferred_element_type=jnp.float32)

        @pl.when(i == 0)
        def _():
            pool_ref[...] = jnp.zeros_like(pool_ref)

        pool_ref[...] += part

        @pl.when(i == nb - 1)
        def _():
            p = pool_ref[...] * (g_ref[0, :] * inv_std)[None, :] \
                + be_ref[0, :][None, :]
            h = lax.dot_general(p, w1_ref[...], (((1,), (0,)), ((), ())),
                                precision=lax.Precision.HIGHEST,
                                preferred_element_type=jnp.float32)
            h = jnp.maximum(h + b1_ref[0, :][None, :], 0.0)
            lg = lax.dot_general(h, w2_ref[...], (((1,), (0,)), ((), ())),
                                 precision=lax.Precision.HIGHEST,
                                 preferred_element_type=jnp.float32)
            lg = lg + b2_ref[0, :][None, :]
            m = jnp.max(lg, axis=1, keepdims=True)
            ex = jnp.exp(lg - m)
            lse = jnp.log(jnp.sum(ex, axis=1, keepdims=True)) + m
            out_ref[...] = lg - lse

    return pl.pallas_call(
        body,
        grid=(nb,),
        in_specs=[pl.BlockSpec((bn, HP), lambda i: (i, 0)),
                  pl.BlockSpec((1, 1, bn), lambda i: (i, 0, 0)),
                  pl.BlockSpec((1, HP), lambda i: (0, 0)),
                  pl.BlockSpec((1, HP), lambda i: (0, 0)),
                  pl.BlockSpec((HP, HP), lambda i: (0, 0)),
                  pl.BlockSpec((1, HP), lambda i: (0, 0)),
                  pl.BlockSpec((HP, 18), lambda i: (0, 0)),
                  pl.BlockSpec((1, 18), lambda i: (0, 0))],
        out_specs=[pl.BlockSpec((G, HP), lambda i: (0, 0)),
                   pl.BlockSpec((G, 18), lambda i: (0, 0))],
        out_shape=[jax.ShapeDtypeStruct((G, HP), jnp.float32),
                   jax.ShapeDtypeStruct((G, 18), jnp.float32)],
    )(o2, batch3, gam, bet, w1, b1, w2, b2)[1]


def _layer_weights(comp, bases, root, bias, fin):
    h = bases.shape[2]
    wf = _tc_small_mm(comp, bases.reshape(NBASE, bases.shape[1] * h))
    w = wf.reshape(R, bases.shape[1], h)
    w = jnp.pad(w, ((0, 0), (0, fin - bases.shape[1]), (0, HP - h)))
    rootp = jnp.pad(root, ((0, fin - root.shape[0]), (0, HP - h)))
    wcat = jnp.concatenate([rootp, w[0], w[1], w[2], w[3]], axis=1)
    bcat = jnp.concatenate([jnp.pad(bias, (0, HP - h)),
                            jnp.zeros((R * HP,), jnp.float32)])
    return wcat, bcat.reshape(1, 5 * HP)


def kernel(x, edge_index, edge_type, batch, comp1, bases1, root1, bias1,
           comp2, bases2, root2, bias2, bn_gamma, bn_beta, fc1_w, fc1_b,
           fc2_w, fc2_b):
    src = edge_index[0]
    dstv = edge_index[1]
    et = edge_type

    hist = _sc_hist(dstv, et)
    gidx, scale = _sc_edge_prep(hist, src, dstv, et)

    wcat1, bcat1 = _layer_weights(comp1, bases1, root1, bias1, FIN)
    z1, hb1 = _tc_matmul(x, wcat1, bcat1, relu_in=False)
    o1 = _sc_agg(hb1.reshape(R * N, HP), z1, gidx, scale, dstv)

    wcat2, bcat2 = _layer_weights(comp2, bases2, root2, bias2, HP)
    z2, hb2 = _tc_matmul(o1, wcat2, bcat2, relu_in=True)
    o2 = _sc_agg(hb2.reshape(R * N, HP), z2, gidx, scale, dstv)

    batch3 = batch.reshape(N // 1000, 1, 1000)
    gam = jnp.pad(bn_gamma, (0, HP - H0)).reshape(1, HP)
    bet = jnp.pad(bn_beta, (0, HP - H0)).reshape(1, HP)
    w1 = jnp.pad(fc1_w, ((0, HP - H0), (0, HP - H0)))
    b1 = jnp.pad(fc1_b, (0, HP - H0)).reshape(1, HP)
    w2 = jnp.pad(fc2_w, ((0, HP - H0), (0, 0)))
    b2 = fc2_b.reshape(1, 18)
    return _tc_pool_head(o2, batch3, gam, bet, w1, b1, w2, b2)


# speedup vs baseline: 13.3796x; 2.2426x over previous
import functools
import math

import jax
import jax.numpy as jnp
from jax import lax
from jax.experimental import pallas as pl
from jax.experimental.pallas import tpu as pltpu
from jax.experimental.pallas import tpu_sc as plsc

N = 50000
E = 800000
R = 4
NBASE = 30
FIN = 86
H0 = 56
G = 32
HP = 64
HH = 32
HIST = 204800
NC, NS = 2, 16
NW = NC * NS
ESH = E // NW
NBATCH = E // 256


def _i32(v):
    return jnp.full((16,), v, jnp.int32)


def _f32(v):
    return jnp.full((16,), v, jnp.float32)


def _mesh():
    return plsc.VectorSubcoreMesh(core_axis_name="c", subcore_axis_name="s")


_SC_PARAMS = pltpu.CompilerParams(use_tc_tiling_on_sc=False)


def _sc_hist(dstv, etype):
    @functools.partial(
        pl.kernel, mesh=_mesh(), compiler_params=_SC_PARAMS,
        out_type=jax.ShapeDtypeStruct((NC, HIST), jnp.float32),
        scratch_types=[
            pltpu.VMEM((2048,), jnp.float32),
            pltpu.VMEM((256,), jnp.int32),
            pltpu.VMEM((256,), jnp.int32),
            pltpu.VMEM((256,), jnp.int32),
            pltpu.VMEM((256,), jnp.float32),
            pltpu.SemaphoreType.DMA,
            pltpu.VMEM_SHARED((HIST,), jnp.float32),
        ],
    )
    def k(dst_hbm, et_hbm, hist_hbm, zbuf, dbuf, tbuf, ibuf, ones, sem,
          hist_sh):
        c = lax.axis_index("c")
        s = lax.axis_index("s")
        wid = s * NC + c

        def zf(i, _):
            zbuf[pl.ds(i * 16, 16)] = _f32(0.0)
            return 0

        lax.fori_loop(0, 128, zf, 0, unroll=False)

        def of(i, _):
            ones[pl.ds(i * 16, 16)] = _f32(1.0)
            return 0

        lax.fori_loop(0, 16, of, 0, unroll=False)

        base = s * 12800
        for j in range(6):
            pltpu.sync_copy(zbuf, hist_sh.at[pl.ds(base + j * 2048, 2048)])
        pltpu.sync_copy(zbuf.at[pl.ds(0, 512)],
                        hist_sh.at[pl.ds(base + 12288, 512)])
        plsc.subcore_barrier()

        eb = wid * ESH

        def batch(b, _):
            off = eb + b * 256
            pltpu.sync_copy(dst_hbm.at[pl.ds(off, 256)], dbuf)
            pltpu.sync_copy(et_hbm.at[pl.ds(off, 256)], tbuf)

            def ch(g, _):
                sl = pl.ds(g * 16, 16)
                ibuf[sl] = tbuf[sl] * _i32(N) + dbuf[sl]
                return 0

            lax.fori_loop(0, 16, ch, 0, unroll=False)
            pltpu.sync_copy(ones, hist_sh.at[ibuf], add=True)
            return 0

        lax.fori_loop(0, 97, batch, 0, unroll=False)

        off = eb + 97 * 256
        pltpu.sync_copy(dst_hbm.at[pl.ds(off, 168)], dbuf.at[pl.ds(0, 168)])
        pltpu.sync_copy(et_hbm.at[pl.ds(off, 168)], tbuf.at[pl.ds(0, 168)])
        lanes = lax.iota(jnp.int32, 16)
        trash = _i32(200000) + lanes + jnp.full((16,), wid * 16, jnp.int32)
        for g in range(11):
            sl = pl.ds(g * 16, 16)
            v = tbuf[sl] * _i32(N) + dbuf[sl]
            if g == 10:
                v = jnp.where(lanes < _i32(8), v, trash)
            ibuf[sl] = v
        for g in range(11, 16):
            ibuf[pl.ds(g * 16, 16)] = trash
        pltpu.sync_copy(ones, hist_sh.at[ibuf], add=True)
        plsc.subcore_barrier()

        for j in range(6):
            pltpu.sync_copy(hist_sh.at[pl.ds(base + j * 2048, 2048)], zbuf)
            pltpu.sync_copy(zbuf, hist_hbm.at[c, pl.ds(base + j * 2048, 2048)])
        pltpu.sync_copy(hist_sh.at[pl.ds(base + 12288, 512)],
                        zbuf.at[pl.ds(0, 512)])
        pltpu.sync_copy(zbuf.at[pl.ds(0, 512)],
                        hist_hbm.at[c, pl.ds(base + 12288, 512)])

    return k(dstv, etype)


def _sc_edge_prep(hist, src, dstv, etype):
    @functools.partial(
        pl.kernel, mesh=_mesh(), compiler_params=_SC_PARAMS,
        out_type=(jax.ShapeDtypeStruct((NBATCH, 2, 256), jnp.int32),
                  jax.ShapeDtypeStruct((E,), jnp.float32)),
        scratch_types=[
            pltpu.VMEM((2048,), jnp.float32),
            pltpu.VMEM((2048,), jnp.float32),
            pltpu.VMEM((256,), jnp.int32),
            pltpu.VMEM((256,), jnp.int32),
            pltpu.VMEM((256,), jnp.int32),
            pltpu.VMEM((256,), jnp.float32),
            pltpu.VMEM((2, 256), jnp.int32),
            pltpu.SemaphoreType.DMA,
            pltpu.VMEM_SHARED((HIST,), jnp.float32),
        ],
    )
    def k(hist_hbm, src_hbm, dst_hbm, et_hbm, meta_hbm, scale_hbm,
          h0buf, h1buf, sbuf, tbuf, ibuf, scbuf, mbuf, sem, recip_sh):
        c = lax.axis_index("c")
        s = lax.axis_index("s")
        wid = s * NC + c
        base = s * 12800
        for j in range(7):
            ln = 2048 if j < 6 else 512
            o = base + j * 2048
            pltpu.sync_copy(hist_hbm.at[0, pl.ds(o, ln)],
                            h0buf.at[pl.ds(0, ln)])
            pltpu.sync_copy(hist_hbm.at[1, pl.ds(o, ln)],
                            h1buf.at[pl.ds(0, ln)])

            def rc(i, _):
                sl = pl.ds(i * 16, 16)
                h = jnp.maximum(h0buf[sl] + h1buf[sl], _f32(1.0))
                h0buf[sl] = _f32(1.0) / h
                return 0

            lax.fori_loop(0, ln // 16, rc, 0, unroll=False)
            pltpu.sync_copy(h0buf.at[pl.ds(0, ln)], recip_sh.at[pl.ds(o, ln)])
        plsc.subcore_barrier()

        bb = wid * 98
        nb3 = jnp.where(wid == NW - 1, NBATCH - 98 * (NW - 1), 98)

        def batch(b, _):
            bidx = bb + b
            off = bidx * 256
            pltpu.sync_copy(src_hbm.at[pl.ds(off, 256)], sbuf)
            pltpu.sync_copy(dst_hbm.at[pl.ds(off, 256)],
                            mbuf.at[1])
            pltpu.sync_copy(et_hbm.at[pl.ds(off, 256)], tbuf)

            def ch(g, _):
                sl = pl.ds(g * 16, 16)
                t = tbuf[sl]
                mbuf[0, sl] = t * _i32(N) + sbuf[sl]
                ibuf[sl] = t * _i32(N) + mbuf[1, sl]
                return 0

            lax.fori_loop(0, 16, ch, 0, unroll=False)
            pltpu.sync_copy(recip_sh.at[ibuf], scbuf)
            pltpu.sync_copy(mbuf, meta_hbm.at[bidx])
            pltpu.sync_copy(scbuf, scale_hbm.at[pl.ds(off, 256)])
            return 0

        lax.fori_loop(0, nb3, batch, 0, unroll=False)

    return k(hist, src, dstv, etype)


def _sc_agg(hbflat, z, meta, scale):
    @functools.partial(
        pl.kernel, mesh=_mesh(), compiler_params=_SC_PARAMS,
        out_type=jax.ShapeDtypeStruct((NC, N, HH), jnp.float32),
        scratch_types=[
            pltpu.VMEM((2, 256), jnp.int32),
            pltpu.VMEM((2, 256), jnp.int32),
            pltpu.VMEM((256,), jnp.float32),
            pltpu.VMEM((256,), jnp.float32),
            pltpu.VMEM((256, HH), jnp.float32),
            pltpu.VMEM((256, HH), jnp.float32),
            pltpu.VMEM((128, HH), jnp.float32),
            pltpu.SemaphoreType.DMA,
            pltpu.SemaphoreType.DMA,
            pltpu.VMEM_SHARED((N, HH), jnp.float32),
        ],
    )
    def k(hb_hbm, z_hbm, meta_hbm, scale_hbm, o_hbm,
          metaA, metaB, scA, scB, rowsA, rowsB, stage, semA, semB, acc_sh):
        c = lax.axis_index("c")
        s = lax.axis_index("s")
        coff = c * (R * N)

        r0 = s * 3125
        for j in range(25):
            ln = 128 if j < 24 else 53
            pltpu.sync_copy(z_hbm.at[c, pl.ds(r0 + j * 128, ln)],
                            stage.at[pl.ds(0, ln)])
            pltpu.sync_copy(stage.at[pl.ds(0, ln)],
                            acc_sh.at[pl.ds(r0 + j * 128, ln)])
        plsc.subcore_barrier()

        bb = s * 195 + jnp.minimum(s, 5)
        nb = jnp.where(s < 5, 196, 195)
        npair = nb // 2

        def adjust_and_fire(mref, rref, sem):
            def adj(g, _):
                sl = pl.ds(g * 16, 16)
                mref[0, sl] = mref[0, sl] + jnp.full((16,), coff, jnp.int32)
                return 0

            lax.fori_loop(0, 16, adj, 0, unroll=False)
            pltpu.async_copy(hb_hbm.at[mref.at[0]], rref, sem)

        def wait_rows(mref, rref, sem):
            pltpu.make_async_copy(hb_hbm.at[mref.at[0]], rref, sem).wait()

        def mul_scatter(mref, scref, rref):
            def mul(g, _):
                sv16 = scref[pl.ds(g * 16, 16)]
                for j in range(16):
                    e = g * 16 + j
                    sv = jnp.full((16,), sv16[j], jnp.float32)
                    rref[e, pl.ds(0, 16)] = rref[e, pl.ds(0, 16)] * sv
                    rref[e, pl.ds(16, 16)] = rref[e, pl.ds(16, 16)] * sv
                return 0

            lax.fori_loop(0, 16, mul, 0, unroll=False)
            pltpu.sync_copy(rref, acc_sh.at[mref.at[1]], add=True)

        pltpu.sync_copy(meta_hbm.at[bb], metaA)
        pltpu.sync_copy(scale_hbm.at[pl.ds(bb * 256, 256)], scA)
        adjust_and_fire(metaA, rowsA, semA)

        def pair(p, _):
            b0 = bb + 2 * p
            pltpu.sync_copy(meta_hbm.at[b0 + 1], metaB)
            pltpu.sync_copy(scale_hbm.at[pl.ds((b0 + 1) * 256, 256)], scB)
            adjust_and_fire(metaB, rowsB, semB)
            wait_rows(metaA, rowsA, semA)
            mul_scatter(metaA, scA, rowsA)
            nxt = jnp.minimum(b0 + 2, NBATCH - 1)
            pltpu.sync_copy(meta_hbm.at[nxt], metaA)
            pltpu.sync_copy(scale_hbm.at[pl.ds(nxt * 256, 256)], scA)
            adjust_and_fire(metaA, rowsA, semA)
            wait_rows(metaB, rowsB, semB)
            mul_scatter(metaB, scB, rowsB)
            return 0

        lax.fori_loop(0, npair, pair, 0, unroll=False)

        wait_rows(metaA, rowsA, semA)

        @pl.when(s >= 5)
        def _():
            mul_scatter(metaA, scA, rowsA)

        plsc.subcore_barrier()
        for j in range(25):
            ln = 128 if j < 24 else 53
            pltpu.sync_copy(acc_sh.at[pl.ds(r0 + j * 128, ln)],
                            stage.at[pl.ds(0, ln)])
            pltpu.sync_copy(stage.at[pl.ds(0, ln)],
                            o_hbm.at[c, pl.ds(r0 + j * 128, ln)])

    return k(hbflat, z, meta, scale)


def _tc_small_mm(a, b):
    def body(a_ref, b_ref, o_ref):
        o_ref[...] = lax.dot_general(
            a_ref[...], b_ref[...], (((1,), (0,)), ((), ())),
            precision=lax.Precision.HIGHEST,
            preferred_element_type=jnp.float32)

    return pl.pallas_call(
        body,
        out_shape=jax.ShapeDtypeStruct((a.shape[0], b.shape[1]), jnp.float32),
    )(a, b)


def _tc_matmul_first(x, wcat, bcat):
    n, fin = x.shape
    bn = 2000
    nb = n // bn

    def body(x_ref, w_ref, b_ref, z_ref, hb_ref):
        y = lax.dot_general(x_ref[...], w_ref[...], (((1,), (0,)), ((), ())),
                            precision=lax.Precision.HIGHEST,
                            preferred_element_type=jnp.float32)
        y = y + b_ref[0, :][None, :]
        for cc in range(NC):
            z_ref[cc] = y[:, HH * cc:HH * (cc + 1)]
            for r in range(R):
                o = HP * (r + 1) + HH * cc
                hb_ref[cc, r] = y[:, o:o + HH]

    return pl.pallas_call(
        body,
        grid=(nb,),
        in_specs=[pl.BlockSpec((bn, fin), lambda i: (i, 0)),
                  pl.BlockSpec((fin, 5 * HP), lambda i: (0, 0)),
                  pl.BlockSpec((1, 5 * HP), lambda i: (0, 0))],
        out_specs=[pl.BlockSpec((NC, bn, HH), lambda i: (0, i, 0)),
                   pl.BlockSpec((NC, R, bn, HH), lambda i: (0, 0, i, 0))],
        out_shape=[jax.ShapeDtypeStruct((NC, n, HH), jnp.float32),
                   jax.ShapeDtypeStruct((NC, R, n, HH), jnp.float32)],
    )(x, wcat, bcat)


def _tc_matmul_pair(xp, wcat, bcat):
    bn = 2000
    nb = N // bn

    def body(x0_ref, x1_ref, w_ref, b_ref, z_ref, hb_ref):
        xb = jnp.concatenate([x0_ref[0], x1_ref[0]], axis=1)
        xb = jnp.maximum(xb, 0.0)
        y = lax.dot_general(xb, w_ref[...], (((1,), (0,)), ((), ())),
                            precision=lax.Precision.HIGHEST,
                            preferred_element_type=jnp.float32)
        y = y + b_ref[0, :][None, :]
        for cc in range(NC):
            z_ref[cc] = y[:, HH * cc:HH * (cc + 1)]
            for r in range(R):
                o = HP * (r + 1) + HH * cc
                hb_ref[cc, r] = y[:, o:o + HH]

    return pl.pallas_call(
        body,
        grid=(nb,),
        in_specs=[pl.BlockSpec((1, bn, HH), lambda i: (0, i, 0)),
                  pl.BlockSpec((1, bn, HH), lambda i: (1, i, 0)),
                  pl.BlockSpec((HP, 5 * HP), lambda i: (0, 0)),
                  pl.BlockSpec((1, 5 * HP), lambda i: (0, 0))],
        out_specs=[pl.BlockSpec((NC, bn, HH), lambda i: (0, i, 0)),
                   pl.BlockSpec((NC, R, bn, HH), lambda i: (0, 0, i, 0))],
        out_shape=[jax.ShapeDtypeStruct((NC, N, HH), jnp.float32),
                   jax.ShapeDtypeStruct((NC, R, N, HH), jnp.float32)],
    )(xp, xp, wcat, bcat)


def _tc_pool_head(o2, batch3, gam, bet, w1, b1, w2, b2):
    bn = 1000
    nb = N // bn
    inv_std = 1.0 / math.sqrt(1.0 + 1e-5)

    def body(x0_ref, x1_ref, bt_ref, g_ref, be_ref, w1_ref, b1_ref, w2_ref,
             b2_ref, pool_ref, out_ref):
        i = pl.program_id(0)
        xb = jnp.concatenate([x0_ref[0], x1_ref[0]], axis=1)
        bb = bt_ref[0, 0, :]
        oh = (lax.broadcasted_iota(jnp.int32, (G, bn), 0)
              == bb[None, :]).astype(jnp.float32)
        part = lax.dot_general(oh, xb, (((1,), (0,)), ((), ())),
                               precision=lax.Precision.HIGHEST,
                               preferred_element_type=jnp.float32)

        @pl.when(i == 0)
        def _():
            pool_ref[...] = jnp.zeros_like(pool_ref)

        pool_ref[...] += part

        @pl.when(i == nb - 1)
        def _():
            p = pool_ref[...] * (g_ref[0, :] * inv_std)[None, :] \
                + be_ref[0, :][None, :]
            h = lax.dot_general(p, w1_ref[...], (((1,), (0,)), ((), ())),
                                precision=lax.Precision.HIGHEST,
                                preferred_element_type=jnp.float32)
            h = jnp.maximum(h + b1_ref[0, :][None, :], 0.0)
            lg = lax.dot_general(h, w2_ref[...], (((1,), (0,)), ((), ())),
                                 precision=lax.Precision.HIGHEST,
                                 preferred_element_type=jnp.float32)
            lg = lg + b2_ref[0, :][None, :]
            m = jnp.max(lg, axis=1, keepdims=True)
            ex = jnp.exp(lg - m)
            lse = jnp.log(jnp.sum(ex, axis=1, keepdims=True)) + m
            out_ref[...] = lg - lse

    return pl.pallas_call(
        body,
        grid=(nb,),
        in_specs=[pl.BlockSpec((1, bn, HH), lambda i: (0, i, 0)),
                  pl.BlockSpec((1, bn, HH), lambda i: (1, i, 0)),
                  pl.BlockSpec((1, 1, bn), lambda i: (i, 0, 0)),
                  pl.BlockSpec((1, HP), lambda i: (0, 0)),
                  pl.BlockSpec((1, HP), lambda i: (0, 0)),
                  pl.BlockSpec((HP, HP), lambda i: (0, 0)),
                  pl.BlockSpec((1, HP), lambda i: (0, 0)),
                  pl.BlockSpec((HP, 18), lambda i: (0, 0)),
                  pl.BlockSpec((1, 18), lambda i: (0, 0))],
        out_specs=[pl.BlockSpec((G, HP), lambda i: (0, 0)),
                   pl.BlockSpec((G, 18), lambda i: (0, 0))],
        out_shape=[jax.ShapeDtypeStruct((G, HP), jnp.float32),
                   jax.ShapeDtypeStruct((G, 18), jnp.float32)],
    )(o2, o2, batch3, gam, bet, w1, b1, w2, b2)[1]


def _layer_weights(comp, bases, root, bias, fin):
    h = bases.shape[2]
    wf = _tc_small_mm(comp, bases.reshape(NBASE, bases.shape[1] * h))
    w = wf.reshape(R, bases.shape[1], h)
    w = jnp.pad(w, ((0, 0), (0, fin - bases.shape[1]), (0, HP - h)))
    rootp = jnp.pad(root, ((0, fin - root.shape[0]), (0, HP - h)))
    wcat = jnp.concatenate([rootp, w[0], w[1], w[2], w[3]], axis=1)
    bcat = jnp.concatenate([jnp.pad(bias, (0, HP - h)),
                            jnp.zeros((R * HP,), jnp.float32)])
    return wcat, bcat.reshape(1, 5 * HP)


def kernel(x, edge_index, edge_type, batch, comp1, bases1, root1, bias1,
           comp2, bases2, root2, bias2, bn_gamma, bn_beta, fc1_w, fc1_b,
           fc2_w, fc2_b):
    src = edge_index[0]
    dstv = edge_index[1]
    et = edge_type

    hist = _sc_hist(dstv, et)
    meta, scale = _sc_edge_prep(hist, src, dstv, et)

    wcat1, bcat1 = _layer_weights(comp1, bases1, root1, bias1, FIN)
    z1, hb1 = _tc_matmul_first(x, wcat1, bcat1)
    o1 = _sc_agg(hb1.reshape(NC * R * N, HH), z1, meta, scale)

    wcat2, bcat2 = _layer_weights(comp2, bases2, root2, bias2, HP)
    z2, hb2 = _tc_matmul_pair(o1, wcat2, bcat2)
    o2 = _sc_agg(hb2.reshape(NC * R * N, HH), z2, meta, scale)

    batch3 = batch.reshape(N // 1000, 1, 1000)
    gam = jnp.pad(bn_gamma, (0, HP - H0)).reshape(1, HP)
    bet = jnp.pad(bn_beta, (0, HP - H0)).reshape(1, HP)
    w1 = jnp.pad(fc1_w, ((0, HP - H0), (0, HP - H0)))
    b1 = jnp.pad(fc1_b, (0, HP - H0)).reshape(1, HP)
    w2 = jnp.pad(fc2_w, ((0, HP - H0), (0, 0)))
    b2 = fc2_b.reshape(1, 18)
    return _tc_pool_head(o2, batch3, gam, bet, w1, b1, w2, b2)
